# Initial kernel scaffold; baseline (speedup 1.0000x reference)
import jax
import jax.numpy as jnp
from jax.experimental import pallas as pl


def kernel(x, edge_index, W1, b1, Wf1, bf1, Wf2, bf2, Wf3, bf3):
    raise NotImplementedError("write your pallas kernel here")


# speedup vs baseline: 60.8740x; 60.8740x over previous
import functools
import jax
import jax.numpy as jnp
from jax import lax
from jax.experimental import pallas as pl
from jax.experimental.pallas import tpu as pltpu
from jax.experimental.pallas import tpu_sc as plsc

N = 100000
E = 1600000
NC, NS = 2, 16
NW = NC * NS
NPAD = 100096
SUB = NPAD // NS
KJ = 16
NT = 25
ROWS_PER_W = KJ * NT
EPW = ROWS_PER_W * 128
E_PAD = EPW * NW
ER = E_PAD // 128
GB = 1000


def _mesh():
    return plsc.VectorSubcoreMesh(
        core_axis_name="c", subcore_axis_name="s",
        num_cores=NC, num_subcores=NS)


_SC_PARAMS = pltpu.CompilerParams(use_tc_tiling_on_sc=False,
                                 needs_layout_passes=False)


def _deg_body(dst_hbm, zeros_hbm, out_hbm, idx_v, ones_v, buf_v, acc_sh):
    cid = lax.axis_index("c")
    sid = lax.axis_index("s")
    wid = sid * NC + cid
    for i in range(128 // 16):
        ones_v[pl.ds(i * 16, 16)] = jnp.full((16,), 1.0, jnp.float32)
    pltpu.sync_copy(zeros_hbm.at[pl.ds(sid * SUB, SUB)], buf_v)
    pltpu.sync_copy(buf_v, acc_sh.at[pl.ds(sid * SUB, SUB)])
    plsc.subcore_barrier()
    row0 = wid * ROWS_PER_W

    def step(t, carry):
        pltpu.sync_copy(dst_hbm.at[pl.ds(row0 + t * KJ, KJ)], idx_v)
        for j in range(KJ):
            pltpu.sync_copy(ones_v, acc_sh.at[idx_v.at[j]], add=True)
        return carry

    lax.fori_loop(0, NT, step, 0)
    plsc.subcore_barrier()
    pltpu.sync_copy(acc_sh.at[pl.ds(sid * SUB, SUB)], buf_v)
    pltpu.sync_copy(buf_v, out_hbm.at[pl.ds(cid * NPAD + sid * SUB, SUB)])


def _deg_call(dst2d, zeros1):
    return pl.kernel(
        _deg_body,
        out_type=jax.ShapeDtypeStruct((NC * NPAD,), jnp.float32),
        mesh=_mesh(),
        scratch_types=[
            pltpu.VMEM((KJ, 128), jnp.int32),
            pltpu.VMEM((128,), jnp.float32),
            pltpu.VMEM((SUB,), jnp.float32),
            pltpu.VMEM_SHARED((NPAD,), jnp.float32),
        ],
        compiler_params=_SC_PARAMS,
    )(dst2d, zeros1)


HALF = NPAD // 2


def _msg_body(src_hbm, dst_hbm, u0_hbm, u1_hbm, zeros_hbm,
              out0_hbm, out1_hbm,
              srcv, dstv, u0h_v, u1h_v, val0_v, val1_v, buf_v,
              acc0_sh, acc1_sh):
    cid = lax.axis_index("c")
    sid = lax.axis_index("s")
    wid = sid * NC + cid
    pltpu.sync_copy(zeros_hbm.at[pl.ds(sid * SUB, SUB)], buf_v)
    pltpu.sync_copy(buf_v, acc0_sh.at[pl.ds(sid * SUB, SUB)])
    pltpu.sync_copy(buf_v, acc1_sh.at[pl.ds(sid * SUB, SUB)])
    plsc.subcore_barrier()
    row0 = wid * ROWS_PER_W

    for p in range(2):
        lo = p * HALF
        pltpu.sync_copy(u0_hbm.at[pl.ds(lo, HALF)], u0h_v)
        pltpu.sync_copy(u1_hbm.at[pl.ds(lo, HALF)], u1h_v)

        def step(t, carry):
            r = row0 + t * KJ
            pltpu.sync_copy(src_hbm.at[pl.ds(r, KJ)], srcv)
            pltpu.sync_copy(dst_hbm.at[pl.ds(r, KJ)], dstv)
            for j in range(KJ):
                for k in range(128 // 16):
                    idx = srcv[j, pl.ds(k * 16, 16)]
                    m = (idx >= lo) & (idx < lo + HALF)
                    lidx = jnp.where(m, idx - lo, 0)
                    v0 = plsc.load_gather(u0h_v, [lidx])
                    v1 = plsc.load_gather(u1h_v, [lidx])
                    zero = jnp.zeros((16,), jnp.float32)
                    val0_v[pl.ds(k * 16, 16)] = jnp.where(m, v0, zero)
                    val1_v[pl.ds(k * 16, 16)] = jnp.where(m, v1, zero)
                pltpu.sync_copy(val0_v, acc0_sh.at[dstv.at[j]], add=True)
                pltpu.sync_copy(val1_v, acc1_sh.at[dstv.at[j]], add=True)
            return carry

        lax.fori_loop(0, NT, step, 0)

    plsc.subcore_barrier()
    pltpu.sync_copy(acc0_sh.at[pl.ds(sid * SUB, SUB)], buf_v)
    pltpu.sync_copy(buf_v, out0_hbm.at[pl.ds(cid * NPAD + sid * SUB, SUB)])
    pltpu.sync_copy(acc1_sh.at[pl.ds(sid * SUB, SUB)], buf_v)
    pltpu.sync_copy(buf_v, out1_hbm.at[pl.ds(cid * NPAD + sid * SUB, SUB)])


def _msg_call(src2d, dst2d, u0, u1, zeros1):
    o = jax.ShapeDtypeStruct((NC * NPAD,), jnp.float32)
    return pl.kernel(
        _msg_body,
        out_type=(o, o),
        mesh=_mesh(),
        scratch_types=[
            pltpu.VMEM((KJ, 128), jnp.int32),
            pltpu.VMEM((KJ, 128), jnp.int32),
            pltpu.VMEM((HALF,), jnp.float32),
            pltpu.VMEM((HALF,), jnp.float32),
            pltpu.VMEM((128,), jnp.float32),
            pltpu.VMEM((128,), jnp.float32),
            pltpu.VMEM((SUB,), jnp.float32),
            pltpu.VMEM_SHARED((NPAD,), jnp.float32),
            pltpu.VMEM_SHARED((NPAD,), jnp.float32),
        ],
        compiler_params=_SC_PARAMS,
    )(src2d, dst2d, u0, u1, zeros1)


def _mid_body(degp_ref, xt_ref, dinv_ref, u0_ref, u1_ref):
    deg = degp_ref[0:1, :] + degp_ref[1:2, :] + 1.0
    dinv = lax.rsqrt(deg)
    dinv_ref[...] = dinv
    u0_ref[...] = dinv * xt_ref[0:1, :]
    u1_ref[...] = dinv * xt_ref[1:2, :]


def _mid_call(degp2, xt):
    o = jax.ShapeDtypeStruct((1, NPAD), jnp.float32)
    return pl.pallas_call(
        _mid_body,
        out_shape=[o, o, o],
    )(degp2, xt)


def _head_body(g0, g1, uu0, uu1, dv, B0, B1, b320, Wf1, bf1, Wf23, bf23,
               fea_out, mu_out, th_out):
    hp = lax.Precision.HIGHEST
    m0 = (g0[...] + uu0[...]) * dv[...]
    m1 = (g1[...] + uu1[...]) * dv[...]
    h = (jnp.dot(m0, B0[...], preferred_element_type=jnp.float32, precision=hp)
         + jnp.dot(m1, B1[...], preferred_element_type=jnp.float32, precision=hp)
         + b320[...])
    h = jnp.maximum(h, 0.0)
    fea = jnp.dot(h, Wf1[...], preferred_element_type=jnp.float32,
                  precision=hp) + bf1[...]
    nrm = jnp.sqrt(jnp.sum(fea * fea, axis=1, keepdims=True))
    fmu = fea / jnp.maximum(nrm, 1e-12)
    s = jnp.dot(fmu, Wf23[...], preferred_element_type=jnp.float32,
                precision=hp) + bf23[...]
    fea_out[...] = fmu
    mu_out[...] = 1.0 / (1.0 + jnp.exp(-s[:, 0:1]))
    sp = s[:, 1:2]
    th_out[...] = jnp.maximum(sp, 0.0) + jnp.log(1.0 + jnp.exp(-jnp.abs(sp)))


def _head_call(g0, g1, uu0, uu1, dv, B0, B1, b320, Wf1, bf1, Wf23, bf23):
    nb = 10000 // GB
    bs_g = pl.BlockSpec((GB, 10), lambda i: (i, 0))

    def full(shape):
        return pl.BlockSpec(shape, lambda i: (0,) * len(shape))

    return pl.pallas_call(
        _head_body,
        grid=(nb,),
        in_specs=[bs_g, bs_g, bs_g, bs_g, bs_g,
                  full((10, 320)), full((10, 320)), full((1, 320)),
                  full((320, 256)), full((1, 256)),
                  full((256, 2)), full((1, 2))],
        out_specs=[pl.BlockSpec((GB, 256), lambda i: (i, 0)),
                   pl.BlockSpec((GB, 1), lambda i: (i, 0)),
                   pl.BlockSpec((GB, 1), lambda i: (i, 0))],
        out_shape=[jax.ShapeDtypeStruct((10000, 256), jnp.float32),
                   jax.ShapeDtypeStruct((10000, 1), jnp.float32),
                   jax.ShapeDtypeStruct((10000, 1), jnp.float32)],
    )(g0, g1, uu0, uu1, dv, B0, B1, b320, Wf1, bf1, Wf23, bf23)


def kernel(x, edge_index, W1, b1, Wf1, bf1, Wf2, bf2, Wf3, bf3):
    src = edge_index[0].astype(jnp.int32)
    dst = edge_index[1].astype(jnp.int32)
    pad = E_PAD - E
    src2d = jnp.concatenate(
        [src, jnp.zeros((pad,), jnp.int32)]).reshape(ER, 128)
    dst2d = jnp.concatenate(
        [dst, jnp.full((pad,), NPAD - 1, jnp.int32)]).reshape(ER, 128)
    zeros1 = jnp.zeros((NPAD,), jnp.float32)

    degp = _deg_call(dst2d, zeros1).reshape(NC, NPAD)
    xt = jnp.pad(x.T, ((0, 0), (0, NPAD - N)))
    dinv, u0, u1 = _mid_call(degp, xt)

    agg0p, agg1p = _msg_call(src2d, dst2d, u0[0], u1[0], zeros1)
    g0 = agg0p.reshape(NC, NPAD).sum(0)[:N].reshape(10000, 10)
    g1 = agg1p.reshape(NC, NPAD).sum(0)[:N].reshape(10000, 10)
    uu0 = u0[0, :N].reshape(10000, 10)
    uu1 = u1[0, :N].reshape(10000, 10)
    dv = dinv[0, :N].reshape(10000, 10)

    B0 = jnp.kron(jnp.eye(10, dtype=jnp.float32), W1[0:1, :])
    B1 = jnp.kron(jnp.eye(10, dtype=jnp.float32), W1[1:2, :])
    b320 = jnp.tile(b1, 10)[None, :]
    Wf23 = jnp.concatenate([Wf2, Wf3], axis=1)
    bf23 = jnp.concatenate([bf2, bf3])[None, :]

    fea_mu, mu, th = _head_call(g0, g1, uu0, uu1, dv,
                                B0, B1, b320, Wf1, bf1[None, :], Wf23, bf23)
    return (fea_mu, mu[:, 0], th[:, 0])


# speedup vs baseline: 82.2874x; 1.3518x over previous
import functools
import jax
import jax.numpy as jnp
from jax import lax
from jax.experimental import pallas as pl
from jax.experimental.pallas import tpu as pltpu
from jax.experimental.pallas import tpu_sc as plsc

N = 100000
E = 1600000
NC, NS = 2, 16
NW = NC * NS
NPAD = 100096
SUB = NPAD // NS
KJ = 16
NT = 25
ROWS_PER_W = KJ * NT
EPW = ROWS_PER_W * 128
E_PAD = EPW * NW
ER = E_PAD // 128
GB = 1000


def _mesh():
    return plsc.VectorSubcoreMesh(
        core_axis_name="c", subcore_axis_name="s",
        num_cores=NC, num_subcores=NS)


_SC_PARAMS = pltpu.CompilerParams(use_tc_tiling_on_sc=False,
                                 needs_layout_passes=False)


def _deg_body(dst_hbm, zeros_hbm, out_hbm, idx_v, ones_v, buf_v, acc_sh):
    cid = lax.axis_index("c")
    sid = lax.axis_index("s")
    wid = sid * NC + cid
    for i in range(128 // 16):
        ones_v[pl.ds(i * 16, 16)] = jnp.full((16,), 1.0, jnp.float32)
    pltpu.sync_copy(zeros_hbm.at[pl.ds(sid * SUB, SUB)], buf_v)
    pltpu.sync_copy(buf_v, acc_sh.at[pl.ds(sid * SUB, SUB)])
    plsc.subcore_barrier()
    row0 = wid * ROWS_PER_W

    def step(t, carry):
        pltpu.sync_copy(dst_hbm.at[pl.ds(row0 + t * KJ, KJ)], idx_v)
        for j in range(KJ):
            pltpu.sync_copy(ones_v, acc_sh.at[idx_v.at[j]], add=True)
        return carry

    lax.fori_loop(0, NT, step, 0)
    plsc.subcore_barrier()
    pltpu.sync_copy(acc_sh.at[pl.ds(sid * SUB, SUB)], buf_v)
    pltpu.sync_copy(buf_v, out_hbm.at[pl.ds(cid * NPAD + sid * SUB, SUB)])


def _deg_call(dst2d, zeros1):
    return pl.kernel(
        _deg_body,
        out_type=jax.ShapeDtypeStruct((NC * NPAD,), jnp.float32),
        mesh=_mesh(),
        scratch_types=[
            pltpu.VMEM((KJ, 128), jnp.int32),
            pltpu.VMEM((128,), jnp.float32),
            pltpu.VMEM((SUB,), jnp.float32),
            pltpu.VMEM_SHARED((NPAD,), jnp.float32),
        ],
        compiler_params=_SC_PARAMS,
    )(dst2d, zeros1)


def _msg_body(src_hbm, dst_hbm, up_hbm, zeros_hbm,
              out0_hbm, out1_hbm,
              srcv, dstv, upt_v, val0_v, val1_v, buf_v,
              acc0_sh, acc1_sh):
    cid = lax.axis_index("c")
    sid = lax.axis_index("s")
    wid = sid * NC + cid
    pltpu.sync_copy(up_hbm, upt_v)
    pltpu.sync_copy(zeros_hbm.at[pl.ds(sid * SUB, SUB)], buf_v)
    pltpu.sync_copy(buf_v, acc0_sh.at[pl.ds(sid * SUB, SUB)])
    pltpu.sync_copy(buf_v, acc1_sh.at[pl.ds(sid * SUB, SUB)])
    plsc.subcore_barrier()
    row0 = wid * ROWS_PER_W

    def step(t, carry):
        r = row0 + t * KJ
        pltpu.sync_copy(src_hbm.at[pl.ds(r, KJ)], srcv)
        pltpu.sync_copy(dst_hbm.at[pl.ds(r, KJ)], dstv)
        for j in range(KJ):
            for k in range(128 // 16):
                idx = srcv[j, pl.ds(k * 16, 16)]
                w = plsc.load_gather(upt_v, [idx])
                bf = plsc.bitcast(w, jnp.bfloat16)
                v0, v1 = plsc.unpack(bf, format=plsc.PackFormat.INTERLEAVED)
                val0_v[pl.ds(k * 16, 16)] = v0
                val1_v[pl.ds(k * 16, 16)] = v1
            pltpu.sync_copy(val0_v, acc0_sh.at[dstv.at[j]], add=True)
            pltpu.sync_copy(val1_v, acc1_sh.at[dstv.at[j]], add=True)
        return carry

    lax.fori_loop(0, NT, step, 0)

    plsc.subcore_barrier()
    pltpu.sync_copy(acc0_sh.at[pl.ds(sid * SUB, SUB)], buf_v)
    pltpu.sync_copy(buf_v, out0_hbm.at[pl.ds(cid * NPAD + sid * SUB, SUB)])
    pltpu.sync_copy(acc1_sh.at[pl.ds(sid * SUB, SUB)], buf_v)
    pltpu.sync_copy(buf_v, out1_hbm.at[pl.ds(cid * NPAD + sid * SUB, SUB)])


def _msg_call(src2d, dst2d, up, zeros1):
    o = jax.ShapeDtypeStruct((NC * NPAD,), jnp.float32)
    return pl.kernel(
        _msg_body,
        out_type=(o, o),
        mesh=_mesh(),
        scratch_types=[
            pltpu.VMEM((KJ, 128), jnp.int32),
            pltpu.VMEM((KJ, 128), jnp.int32),
            pltpu.VMEM((NPAD,), jnp.int32),
            pltpu.VMEM((128,), jnp.float32),
            pltpu.VMEM((128,), jnp.float32),
            pltpu.VMEM((SUB,), jnp.float32),
            pltpu.VMEM_SHARED((NPAD,), jnp.float32),
            pltpu.VMEM_SHARED((NPAD,), jnp.float32),
        ],
        compiler_params=_SC_PARAMS,
    )(src2d, dst2d, up, zeros1)


def _mid_body(degp_ref, xt_ref, dinv_ref, u0_ref, u1_ref, up_ref):
    deg = degp_ref[0:1, :] + degp_ref[1:2, :] + 1.0
    dinv = lax.rsqrt(deg)
    u0 = dinv * xt_ref[0:1, :]
    u1 = dinv * xt_ref[1:2, :]
    dinv_ref[...] = dinv
    u0_ref[...] = u0
    u1_ref[...] = u1
    b0 = lax.bitcast_convert_type(
        u0.astype(jnp.bfloat16), jnp.uint16).astype(jnp.uint32)
    b1 = lax.bitcast_convert_type(
        u1.astype(jnp.bfloat16), jnp.uint16).astype(jnp.uint32)
    up_ref[...] = lax.bitcast_convert_type(b0 | (b1 << 16), jnp.int32)


def _mid_call(degp2, xt):
    o = jax.ShapeDtypeStruct((1, NPAD), jnp.float32)
    oi = jax.ShapeDtypeStruct((1, NPAD), jnp.int32)
    return pl.pallas_call(
        _mid_body,
        out_shape=[o, o, o, oi],
    )(degp2, xt)


def _head_body(g0, g1, uu0, uu1, dv, B0, B1, b320, Wf1, bf1, Wf23, bf23,
               fea_out, mu_out, th_out):
    hp = lax.Precision.HIGHEST
    m0 = (g0[...] + uu0[...]) * dv[...]
    m1 = (g1[...] + uu1[...]) * dv[...]
    h = (jnp.dot(m0, B0[...], preferred_element_type=jnp.float32, precision=hp)
         + jnp.dot(m1, B1[...], preferred_element_type=jnp.float32, precision=hp)
         + b320[...])
    h = jnp.maximum(h, 0.0)
    fea = jnp.dot(h, Wf1[...], preferred_element_type=jnp.float32,
                  precision=hp) + bf1[...]
    nrm = jnp.sqrt(jnp.sum(fea * fea, axis=1, keepdims=True))
    fmu = fea / jnp.maximum(nrm, 1e-12)
    s = jnp.dot(fmu, Wf23[...], preferred_element_type=jnp.float32,
                precision=hp) + bf23[...]
    fea_out[...] = fmu
    mu_out[...] = 1.0 / (1.0 + jnp.exp(-s[:, 0:1]))
    sp = s[:, 1:2]
    th_out[...] = jnp.maximum(sp, 0.0) + jnp.log(1.0 + jnp.exp(-jnp.abs(sp)))


def _head_call(g0, g1, uu0, uu1, dv, B0, B1, b320, Wf1, bf1, Wf23, bf23):
    nb = 10000 // GB
    bs_g = pl.BlockSpec((GB, 10), lambda i: (i, 0))

    def full(shape):
        return pl.BlockSpec(shape, lambda i: (0,) * len(shape))

    return pl.pallas_call(
        _head_body,
        grid=(nb,),
        in_specs=[bs_g, bs_g, bs_g, bs_g, bs_g,
                  full((10, 320)), full((10, 320)), full((1, 320)),
                  full((320, 256)), full((1, 256)),
                  full((256, 2)), full((1, 2))],
        out_specs=[pl.BlockSpec((GB, 256), lambda i: (i, 0)),
                   pl.BlockSpec((GB, 1), lambda i: (i, 0)),
                   pl.BlockSpec((GB, 1), lambda i: (i, 0))],
        out_shape=[jax.ShapeDtypeStruct((10000, 256), jnp.float32),
                   jax.ShapeDtypeStruct((10000, 1), jnp.float32),
                   jax.ShapeDtypeStruct((10000, 1), jnp.float32)],
    )(g0, g1, uu0, uu1, dv, B0, B1, b320, Wf1, bf1, Wf23, bf23)


def kernel(x, edge_index, W1, b1, Wf1, bf1, Wf2, bf2, Wf3, bf3):
    src = edge_index[0].astype(jnp.int32)
    dst = edge_index[1].astype(jnp.int32)
    pad = E_PAD - E
    src2d = jnp.concatenate(
        [src, jnp.zeros((pad,), jnp.int32)]).reshape(ER, 128)
    dst2d = jnp.concatenate(
        [dst, jnp.full((pad,), NPAD - 1, jnp.int32)]).reshape(ER, 128)
    zeros1 = jnp.zeros((NPAD,), jnp.float32)

    degp = _deg_call(dst2d, zeros1).reshape(NC, NPAD)
    xt = jnp.pad(x.T, ((0, 0), (0, NPAD - N)))
    dinv, u0, u1, up = _mid_call(degp, xt)

    agg0p, agg1p = _msg_call(src2d, dst2d, up[0], zeros1)
    g0 = agg0p.reshape(NC, NPAD).sum(0)[:N].reshape(10000, 10)
    g1 = agg1p.reshape(NC, NPAD).sum(0)[:N].reshape(10000, 10)
    uu0 = u0[0, :N].reshape(10000, 10)
    uu1 = u1[0, :N].reshape(10000, 10)
    dv = dinv[0, :N].reshape(10000, 10)

    B0 = jnp.kron(jnp.eye(10, dtype=jnp.float32), W1[0:1, :])
    B1 = jnp.kron(jnp.eye(10, dtype=jnp.float32), W1[1:2, :])
    b320 = jnp.tile(b1, 10)[None, :]
    Wf23 = jnp.concatenate([Wf2, Wf3], axis=1)
    bf23 = jnp.concatenate([bf2, bf3])[None, :]

    fea_mu, mu, th = _head_call(g0, g1, uu0, uu1, dv,
                                B0, B1, b320, Wf1, bf1[None, :], Wf23, bf23)
    return (fea_mu, mu[:, 0], th[:, 0])


# speedup vs baseline: 99.7194x; 1.2118x over previous
import functools
import jax
import jax.numpy as jnp
from jax import lax
from jax.experimental import pallas as pl
from jax.experimental.pallas import tpu as pltpu
from jax.experimental.pallas import tpu_sc as plsc

N = 100000
E = 1600000
NC, NS = 2, 16
NW = NC * NS
NPAD = 100096
SUB = NPAD // NS
KJ = 16
ER = E // 128
NCH = ER // KJ
TAIL = ER - NCH * KJ
BASE_CH = NCH // NW
EXTRA = NCH - BASE_CH * NW
GB = 1000


def _worker_chunks(wid):
    c0 = wid * BASE_CH + jnp.minimum(wid, EXTRA)
    nch = BASE_CH + (wid < EXTRA).astype(jnp.int32)
    return c0, nch


def _mesh():
    return plsc.VectorSubcoreMesh(
        core_axis_name="c", subcore_axis_name="s",
        num_cores=NC, num_subcores=NS)


_SC_PARAMS = pltpu.CompilerParams(use_tc_tiling_on_sc=False,
                                 needs_layout_passes=False)


def _deg_body(dst_hbm, zeros_hbm, out_hbm, idx_v, ones_v, buf_v, acc_sh):
    cid = lax.axis_index("c")
    sid = lax.axis_index("s")
    wid = sid * NC + cid
    for i in range(128 // 16):
        ones_v[pl.ds(i * 16, 16)] = jnp.full((16,), 1.0, jnp.float32)
    pltpu.sync_copy(zeros_hbm.at[pl.ds(sid * SUB, SUB)], buf_v)
    pltpu.sync_copy(buf_v, acc_sh.at[pl.ds(sid * SUB, SUB)])
    plsc.subcore_barrier()
    c0, nch = _worker_chunks(wid)

    def step(t, carry):
        pltpu.sync_copy(dst_hbm.at[pl.ds((c0 + t) * KJ, KJ)], idx_v)
        for j in range(KJ):
            pltpu.sync_copy(ones_v, acc_sh.at[idx_v.at[j]], add=True)
        return carry

    lax.fori_loop(0, nch, step, 0)

    @pl.when(wid == NW - 1)
    def _tail():
        pltpu.sync_copy(dst_hbm.at[pl.ds(NCH * KJ, TAIL)],
                        idx_v.at[pl.ds(0, TAIL)])
        for j in range(TAIL):
            pltpu.sync_copy(ones_v, acc_sh.at[idx_v.at[j]], add=True)

    plsc.subcore_barrier()
    pltpu.sync_copy(acc_sh.at[pl.ds(sid * SUB, SUB)], buf_v)
    pltpu.sync_copy(buf_v, out_hbm.at[pl.ds(cid * NPAD + sid * SUB, SUB)])


def _deg_call(dst2d, zeros1):
    return pl.kernel(
        _deg_body,
        out_type=jax.ShapeDtypeStruct((NC * NPAD,), jnp.float32),
        mesh=_mesh(),
        scratch_types=[
            pltpu.VMEM((KJ, 128), jnp.int32),
            pltpu.VMEM((128,), jnp.float32),
            pltpu.VMEM((SUB,), jnp.float32),
            pltpu.VMEM_SHARED((NPAD,), jnp.float32),
        ],
        compiler_params=_SC_PARAMS,
    )(dst2d, zeros1)


def _msg_body(src_hbm, dst_hbm, up_hbm, zeros_hbm,
              out0_hbm, out1_hbm,
              srcv, dstv, upt_v, val0_v, val1_v, buf_v,
              acc0_sh, acc1_sh):
    cid = lax.axis_index("c")
    sid = lax.axis_index("s")
    wid = sid * NC + cid
    pltpu.sync_copy(up_hbm, upt_v)
    pltpu.sync_copy(zeros_hbm.at[pl.ds(sid * SUB, SUB)], buf_v)
    pltpu.sync_copy(buf_v, acc0_sh.at[pl.ds(sid * SUB, SUB)])
    pltpu.sync_copy(buf_v, acc1_sh.at[pl.ds(sid * SUB, SUB)])
    plsc.subcore_barrier()
    c0, nch = _worker_chunks(wid)

    def _do_row(j, srcv_row, dstv_row):
        for k in range(128 // 16):
            idx = srcv_row[j, pl.ds(k * 16, 16)]
            w = plsc.load_gather(upt_v, [idx])
            bf = plsc.bitcast(w, jnp.bfloat16)
            v0, v1 = plsc.unpack(bf, format=plsc.PackFormat.INTERLEAVED)
            val0_v[pl.ds(k * 16, 16)] = v0
            val1_v[pl.ds(k * 16, 16)] = v1
        pltpu.sync_copy(val0_v, acc0_sh.at[dstv_row.at[j]], add=True)
        pltpu.sync_copy(val1_v, acc1_sh.at[dstv_row.at[j]], add=True)

    def step(t, carry):
        r = (c0 + t) * KJ
        pltpu.sync_copy(src_hbm.at[pl.ds(r, KJ)], srcv)
        pltpu.sync_copy(dst_hbm.at[pl.ds(r, KJ)], dstv)
        for j in range(KJ):
            _do_row(j, srcv, dstv)
        return carry

    lax.fori_loop(0, nch, step, 0)

    @pl.when(wid == NW - 1)
    def _tail():
        pltpu.sync_copy(src_hbm.at[pl.ds(NCH * KJ, TAIL)],
                        srcv.at[pl.ds(0, TAIL)])
        pltpu.sync_copy(dst_hbm.at[pl.ds(NCH * KJ, TAIL)],
                        dstv.at[pl.ds(0, TAIL)])
        for j in range(TAIL):
            _do_row(j, srcv, dstv)

    plsc.subcore_barrier()
    pltpu.sync_copy(acc0_sh.at[pl.ds(sid * SUB, SUB)], buf_v)
    pltpu.sync_copy(buf_v, out0_hbm.at[pl.ds(cid * NPAD + sid * SUB, SUB)])
    pltpu.sync_copy(acc1_sh.at[pl.ds(sid * SUB, SUB)], buf_v)
    pltpu.sync_copy(buf_v, out1_hbm.at[pl.ds(cid * NPAD + sid * SUB, SUB)])


def _msg_call(src2d, dst2d, up, zeros1):
    o = jax.ShapeDtypeStruct((NC * NPAD,), jnp.float32)
    return pl.kernel(
        _msg_body,
        out_type=(o, o),
        mesh=_mesh(),
        scratch_types=[
            pltpu.VMEM((KJ, 128), jnp.int32),
            pltpu.VMEM((KJ, 128), jnp.int32),
            pltpu.VMEM((NPAD,), jnp.int32),
            pltpu.VMEM((128,), jnp.float32),
            pltpu.VMEM((128,), jnp.float32),
            pltpu.VMEM((SUB,), jnp.float32),
            pltpu.VMEM_SHARED((NPAD,), jnp.float32),
            pltpu.VMEM_SHARED((NPAD,), jnp.float32),
        ],
        compiler_params=_SC_PARAMS,
    )(src2d, dst2d, up, zeros1)


def _mid_body(degp_ref, xt_ref, dinv_ref, u0_ref, u1_ref, up_ref):
    deg = degp_ref[0:1, :] + degp_ref[1:2, :] + 1.0
    dinv = lax.rsqrt(deg)
    u0 = dinv * xt_ref[0:1, :]
    u1 = dinv * xt_ref[1:2, :]
    dinv_ref[...] = dinv
    u0_ref[...] = u0
    u1_ref[...] = u1
    b0 = lax.bitcast_convert_type(
        u0.astype(jnp.bfloat16), jnp.uint16).astype(jnp.uint32)
    b1 = lax.bitcast_convert_type(
        u1.astype(jnp.bfloat16), jnp.uint16).astype(jnp.uint32)
    up_ref[...] = lax.bitcast_convert_type(b0 | (b1 << 16), jnp.int32)


def _mid_call(degp2, xt):
    o = jax.ShapeDtypeStruct((1, NPAD), jnp.float32)
    oi = jax.ShapeDtypeStruct((1, NPAD), jnp.int32)
    return pl.pallas_call(
        _mid_body,
        out_shape=[o, o, o, oi],
    )(degp2, xt)


def _head_body(m0, m1, B0, B1, b320, Wf1, bf1, Wf23, bf23,
               fea_out, mu_out, th_out):
    h = (jnp.dot(m0[...], B0[...], preferred_element_type=jnp.float32)
         + jnp.dot(m1[...], B1[...], preferred_element_type=jnp.float32)
         + b320[...])
    h = jnp.maximum(h, 0.0)
    fea = jnp.dot(h, Wf1[...], preferred_element_type=jnp.float32) + bf1[...]
    nrm = jnp.sqrt(jnp.sum(fea * fea, axis=1, keepdims=True))
    fmu = fea / jnp.maximum(nrm, 1e-12)
    s = jnp.dot(fmu, Wf23[...], preferred_element_type=jnp.float32) + bf23[...]
    fea_out[...] = fmu
    mu_out[...] = 1.0 / (1.0 + jnp.exp(-s[:, 0:1]))
    sp = s[:, 1:2]
    th_out[...] = jnp.maximum(sp, 0.0) + jnp.log(1.0 + jnp.exp(-jnp.abs(sp)))


def _head_call(m0, m1, B0, B1, b320, Wf1, bf1, Wf23, bf23):
    nb = 10000 // GB
    bs_g = pl.BlockSpec((GB, 10), lambda i: (i, 0))

    def full(shape):
        return pl.BlockSpec(shape, lambda i: (0,) * len(shape))

    return pl.pallas_call(
        _head_body,
        grid=(nb,),
        in_specs=[bs_g, bs_g,
                  full((10, 320)), full((10, 320)), full((1, 320)),
                  full((320, 256)), full((1, 256)),
                  full((256, 2)), full((1, 2))],
        out_specs=[pl.BlockSpec((GB, 256), lambda i: (i, 0)),
                   pl.BlockSpec((GB, 1), lambda i: (i, 0)),
                   pl.BlockSpec((GB, 1), lambda i: (i, 0))],
        out_shape=[jax.ShapeDtypeStruct((10000, 256), jnp.float32),
                   jax.ShapeDtypeStruct((10000, 1), jnp.float32),
                   jax.ShapeDtypeStruct((10000, 1), jnp.float32)],
    )(m0, m1, B0, B1, b320, Wf1, bf1, Wf23, bf23)


def kernel(x, edge_index, W1, b1, Wf1, bf1, Wf2, bf2, Wf3, bf3):
    src2d = edge_index[0].astype(jnp.int32).reshape(ER, 128)
    dst2d = edge_index[1].astype(jnp.int32).reshape(ER, 128)
    zeros1 = jnp.zeros((NPAD,), jnp.float32)

    degp = _deg_call(dst2d, zeros1).reshape(NC, NPAD)
    xt = jnp.pad(x.T, ((0, 0), (0, NPAD - N)))
    dinv, u0, u1, up = _mid_call(degp, xt)

    agg0p, agg1p = _msg_call(src2d, dst2d, up[0], zeros1)
    m0 = ((agg0p.reshape(NC, NPAD).sum(0) + u0[0])
          * dinv[0])[:N].reshape(10000, 10)
    m1 = ((agg1p.reshape(NC, NPAD).sum(0) + u1[0])
          * dinv[0])[:N].reshape(10000, 10)

    B0 = jnp.kron(jnp.eye(10, dtype=jnp.float32), W1[0:1, :])
    B1 = jnp.kron(jnp.eye(10, dtype=jnp.float32), W1[1:2, :])
    b320 = jnp.tile(b1, 10)[None, :]
    Wf23 = jnp.concatenate([Wf2, Wf3], axis=1)
    bf23 = jnp.concatenate([bf2, bf3])[None, :]

    fea_mu, mu, th = _head_call(m0, m1, B0, B1, b320,
                                Wf1, bf1[None, :], Wf23, bf23)
    return (fea_mu, mu[:, 0], th[:, 0])


# speedup vs baseline: 191.4255x; 1.9196x over previous
import functools
import jax
import jax.numpy as jnp
from jax import lax
from jax.experimental import pallas as pl
from jax.experimental.pallas import tpu as pltpu
from jax.experimental.pallas import tpu_sc as plsc

N = 100000
E = 1600000
NC, NS = 2, 16
NW = NC * NS
NPAD = 100096
SUB = NPAD // NS
KJ = 16
ER = E // 128
NCH = ER // KJ
TAIL = ER - NCH * KJ
BASE_CH = NCH // NW
EXTRA = NCH - BASE_CH * NW
GB = 1000


def _worker_chunks(wid):
    c0 = wid * BASE_CH + jnp.minimum(wid, EXTRA)
    nch = BASE_CH + (wid < EXTRA).astype(jnp.int32)
    return c0, nch


def _mesh():
    return plsc.VectorSubcoreMesh(
        core_axis_name="c", subcore_axis_name="s",
        num_cores=NC, num_subcores=NS)


_SC_PARAMS = pltpu.CompilerParams(use_tc_tiling_on_sc=False,
                                 needs_layout_passes=False)


def _deg_body(e3_hbm, zeros_hbm, out_hbm, ev, ones_v, buf_v, acc_sh, sem):
    cid = lax.axis_index("c")
    sid = lax.axis_index("s")
    wid = sid * NC + cid
    for i in range(128 // 16):
        ones_v[pl.ds(i * 16, 16)] = jnp.full((16,), 1.0, jnp.float32)
    pltpu.sync_copy(zeros_hbm.at[pl.ds(sid * SUB, SUB)], buf_v)
    pltpu.sync_copy(buf_v, acc_sh.at[pl.ds(sid * SUB, SUB)])
    plsc.subcore_barrier()
    c0, nch = _worker_chunks(wid)

    def step(t, carry):
        pltpu.sync_copy(e3_hbm.at[pl.ds((c0 + t) * KJ, KJ)], ev)
        descs = []
        for j in range(KJ):
            descs.append(pltpu.async_copy(
                ones_v, acc_sh.at[ev.at[j, 1]], sem, add=True))
        for d in descs:
            d.wait()
        return carry

    lax.fori_loop(0, nch, step, 0)

    @pl.when(wid == NW - 1)
    def _tail():
        pltpu.sync_copy(e3_hbm.at[pl.ds(NCH * KJ, TAIL)],
                        ev.at[pl.ds(0, TAIL)])
        for j in range(TAIL):
            pltpu.sync_copy(ones_v, acc_sh.at[ev.at[j, 1]], add=True)

    plsc.subcore_barrier()
    pltpu.sync_copy(acc_sh.at[pl.ds(sid * SUB, SUB)], buf_v)
    pltpu.sync_copy(buf_v, out_hbm.at[pl.ds(cid * NPAD + sid * SUB, SUB)])


def _deg_call(e3, zeros1):
    return pl.kernel(
        _deg_body,
        out_type=jax.ShapeDtypeStruct((NC * NPAD,), jnp.float32),
        mesh=_mesh(),
        scratch_types=[
            pltpu.VMEM((KJ, 2, 128), jnp.int32),
            pltpu.VMEM((128,), jnp.float32),
            pltpu.VMEM((SUB,), jnp.float32),
            pltpu.VMEM_SHARED((NPAD,), jnp.float32),
            pltpu.SemaphoreType.DMA,
        ],
        compiler_params=_SC_PARAMS,
    )(e3, zeros1)


def _msg_body(e3_hbm, up_hbm, zeros_hbm,
              out0_hbm, out1_hbm,
              ev, upt_v, val0_v, val1_v, buf_v,
              acc0_sh, acc1_sh, sem):
    cid = lax.axis_index("c")
    sid = lax.axis_index("s")
    wid = sid * NC + cid
    pltpu.sync_copy(up_hbm, upt_v)
    pltpu.sync_copy(zeros_hbm.at[pl.ds(sid * SUB, SUB)], buf_v)
    pltpu.sync_copy(buf_v, acc0_sh.at[pl.ds(sid * SUB, SUB)])
    pltpu.sync_copy(buf_v, acc1_sh.at[pl.ds(sid * SUB, SUB)])
    plsc.subcore_barrier()
    c0, nch = _worker_chunks(wid)

    def _gather_row(j):
        for k in range(128 // 16):
            idx = ev[j, 0, pl.ds(k * 16, 16)]
            w = plsc.load_gather(upt_v, [idx])
            bf = plsc.bitcast(w, jnp.bfloat16)
            v0, v1 = plsc.unpack(bf, format=plsc.PackFormat.INTERLEAVED)
            val0_v[j, pl.ds(k * 16, 16)] = v0
            val1_v[j, pl.ds(k * 16, 16)] = v1

    def step(t, carry):
        pltpu.sync_copy(e3_hbm.at[pl.ds((c0 + t) * KJ, KJ)], ev)
        descs = []
        for j in range(KJ):
            _gather_row(j)
            descs.append(pltpu.async_copy(
                val0_v.at[j], acc0_sh.at[ev.at[j, 1]], sem, add=True))
            descs.append(pltpu.async_copy(
                val1_v.at[j], acc1_sh.at[ev.at[j, 1]], sem, add=True))
        for d in descs:
            d.wait()
        return carry

    lax.fori_loop(0, nch, step, 0)

    @pl.when(wid == NW - 1)
    def _tail():
        pltpu.sync_copy(e3_hbm.at[pl.ds(NCH * KJ, TAIL)],
                        ev.at[pl.ds(0, TAIL)])
        for j in range(TAIL):
            _gather_row(j)
            pltpu.sync_copy(val0_v.at[j], acc0_sh.at[ev.at[j, 1]], add=True)
            pltpu.sync_copy(val1_v.at[j], acc1_sh.at[ev.at[j, 1]], add=True)

    plsc.subcore_barrier()
    pltpu.sync_copy(acc0_sh.at[pl.ds(sid * SUB, SUB)], buf_v)
    pltpu.sync_copy(buf_v, out0_hbm.at[pl.ds(cid * NPAD + sid * SUB, SUB)])
    pltpu.sync_copy(acc1_sh.at[pl.ds(sid * SUB, SUB)], buf_v)
    pltpu.sync_copy(buf_v, out1_hbm.at[pl.ds(cid * NPAD + sid * SUB, SUB)])


def _msg_call(e3, up, zeros1):
    o = jax.ShapeDtypeStruct((NC * NPAD,), jnp.float32)
    return pl.kernel(
        _msg_body,
        out_type=(o, o),
        mesh=_mesh(),
        scratch_types=[
            pltpu.VMEM((KJ, 2, 128), jnp.int32),
            pltpu.VMEM((NPAD,), jnp.int32),
            pltpu.VMEM((KJ, 128), jnp.float32),
            pltpu.VMEM((KJ, 128), jnp.float32),
            pltpu.VMEM((SUB,), jnp.float32),
            pltpu.VMEM_SHARED((NPAD,), jnp.float32),
            pltpu.VMEM_SHARED((NPAD,), jnp.float32),
            pltpu.SemaphoreType.DMA,
        ],
        compiler_params=_SC_PARAMS,
    )(e3, up, zeros1)


def _mid_body(degp_ref, xt_ref, dinv_ref, u0_ref, u1_ref, up_ref):
    deg = degp_ref[0:1, :] + degp_ref[1:2, :] + 1.0
    dinv = lax.rsqrt(deg)
    u0 = dinv * xt_ref[0:1, :]
    u1 = dinv * xt_ref[1:2, :]
    dinv_ref[...] = dinv
    u0_ref[...] = u0
    u1_ref[...] = u1
    b0 = lax.bitcast_convert_type(
        u0.astype(jnp.bfloat16), jnp.uint16).astype(jnp.uint32)
    b1 = lax.bitcast_convert_type(
        u1.astype(jnp.bfloat16), jnp.uint16).astype(jnp.uint32)
    up_ref[...] = lax.bitcast_convert_type(b0 | (b1 << 16), jnp.int32)


def _mid_call(degp2, xt):
    o = jax.ShapeDtypeStruct((1, NPAD), jnp.float32)
    oi = jax.ShapeDtypeStruct((1, NPAD), jnp.int32)
    return pl.pallas_call(
        _mid_body,
        out_shape=[o, o, o, oi],
    )(degp2, xt)


def _head_body(m0, m1, B0, B1, b320, Wf1, bf1, Wf23, bf23,
               fea_out, mu_out, th_out):
    h = (jnp.dot(m0[...], B0[...], preferred_element_type=jnp.float32)
         + jnp.dot(m1[...], B1[...], preferred_element_type=jnp.float32)
         + b320[...])
    h = jnp.maximum(h, 0.0)
    fea = jnp.dot(h, Wf1[...], preferred_element_type=jnp.float32) + bf1[...]
    nrm = jnp.sqrt(jnp.sum(fea * fea, axis=1, keepdims=True))
    fmu = fea / jnp.maximum(nrm, 1e-12)
    s = jnp.dot(fmu, Wf23[...], preferred_element_type=jnp.float32) + bf23[...]
    fea_out[...] = fmu
    mu_out[...] = 1.0 / (1.0 + jnp.exp(-s[:, 0:1]))
    sp = s[:, 1:2]
    th_out[...] = jnp.maximum(sp, 0.0) + jnp.log(1.0 + jnp.exp(-jnp.abs(sp)))


def _head_call(m0, m1, B0, B1, b320, Wf1, bf1, Wf23, bf23):
    nb = 10000 // GB
    bs_g = pl.BlockSpec((GB, 10), lambda i: (i, 0))

    def full(shape):
        return pl.BlockSpec(shape, lambda i: (0,) * len(shape))

    return pl.pallas_call(
        _head_body,
        grid=(nb,),
        in_specs=[bs_g, bs_g,
                  full((10, 320)), full((10, 320)), full((1, 320)),
                  full((320, 256)), full((1, 256)),
                  full((256, 2)), full((1, 2))],
        out_specs=[pl.BlockSpec((GB, 256), lambda i: (i, 0)),
                   pl.BlockSpec((GB, 1), lambda i: (i, 0)),
                   pl.BlockSpec((GB, 1), lambda i: (i, 0))],
        out_shape=[jax.ShapeDtypeStruct((10000, 256), jnp.float32),
                   jax.ShapeDtypeStruct((10000, 1), jnp.float32),
                   jax.ShapeDtypeStruct((10000, 1), jnp.float32)],
    )(m0, m1, B0, B1, b320, Wf1, bf1, Wf23, bf23)


def kernel(x, edge_index, W1, b1, Wf1, bf1, Wf2, bf2, Wf3, bf3):
    e3 = edge_index.astype(jnp.int32).reshape(2, ER, 128).transpose(1, 0, 2)
    zeros1 = jnp.zeros((NPAD,), jnp.float32)

    degp = _deg_call(e3, zeros1).reshape(NC, NPAD)
    xt = jnp.pad(x.T, ((0, 0), (0, NPAD - N)))
    dinv, u0, u1, up = _mid_call(degp, xt)

    agg0p, agg1p = _msg_call(e3, up[0], zeros1)
    m0 = ((agg0p.reshape(NC, NPAD).sum(0) + u0[0])
          * dinv[0])[:N].reshape(10000, 10)
    m1 = ((agg1p.reshape(NC, NPAD).sum(0) + u1[0])
          * dinv[0])[:N].reshape(10000, 10)

    B0 = jnp.kron(jnp.eye(10, dtype=jnp.float32), W1[0:1, :])
    B1 = jnp.kron(jnp.eye(10, dtype=jnp.float32), W1[1:2, :])
    b320 = jnp.tile(b1, 10)[None, :]
    Wf23 = jnp.concatenate([Wf2, Wf3], axis=1)
    bf23 = jnp.concatenate([bf2, bf3])[None, :]

    fea_mu, mu, th = _head_call(m0, m1, B0, B1, b320,
                                Wf1, bf1[None, :], Wf23, bf23)
    return (fea_mu, mu[:, 0], th[:, 0])
